# Initial kernel scaffold; baseline (speedup 1.0000x reference)
import jax
import jax.numpy as jnp
from jax.experimental import pallas as pl


def kernel(student_id, exercise_id, theta_table, disc_table, diff_table, guess_table):
    raise NotImplementedError("write your pallas kernel here")


# speedup vs baseline: 1.1641x; 1.1641x over previous
import functools

import jax
import jax.numpy as jnp
from jax import lax
from jax.experimental import pallas as pl
from jax.experimental.pallas import tpu as pltpu
from jax.experimental.pallas import tpu_sc as plsc

D = 1.702
NC = 2
NS = 16
NW = NC * NS
L = 16
B = 16384
BPW = B // NW
IDXW = 128
NCHUNK = BPW // IDXW


def _irt_body(sid_hbm, eid_hbm, theta_hbm, disc_hbm, diff_hbm, guess_hbm,
              out_hbm, sidx, eidx, th, di, df, gu, out_v, sem):
    wid = lax.axis_index("s") * NC + lax.axis_index("c")
    pltpu.sync_copy(sid_hbm.at[pl.ds(wid * NCHUNK, NCHUNK)], sidx)
    pltpu.sync_copy(eid_hbm.at[pl.ds(wid * NCHUNK, NCHUNK)], eidx)
    copies = []
    for j in range(NCHUNK):
        dst = pl.ds(j * IDXW, IDXW)
        copies.append(pltpu.async_copy(theta_hbm.at[sidx.at[j]], th.at[dst], sem))
        copies.append(pltpu.async_copy(disc_hbm.at[eidx.at[j]], di.at[dst], sem))
        copies.append(pltpu.async_copy(diff_hbm.at[eidx.at[j]], df.at[dst], sem))
        copies.append(pltpu.async_copy(guess_hbm.at[eidx.at[j]], gu.at[dst], sem))
    for c in copies:
        c.wait()
    for i in range(BPW // L):
        sl = pl.ds(i * L, L)
        x = di[sl]
        u = jnp.exp(-jnp.abs(x))
        s = u / (2.0 + u)
        s2 = s * s
        poly = 1.0 + s2 * (1.0 / 3.0 + s2 * (1.0 / 5.0 + s2 * (1.0 / 7.0 + s2 * (1.0 / 9.0))))
        sp = jnp.maximum(x, 0.0) + 2.0 * s * poly
        g = gu[sl]
        e = jnp.exp(-D * sp * (th[sl] - df[sl]))
        out_v[sl] = g + (1.0 - g) / (1.0 + e)
    pltpu.sync_copy(out_v, out_hbm.at[pl.ds(wid * BPW, BPW)])


_irt = functools.partial(
    pl.kernel,
    out_type=jax.ShapeDtypeStruct((B,), jnp.float32),
    mesh=plsc.VectorSubcoreMesh(core_axis_name="c", subcore_axis_name="s"),
    scratch_types=[
        pltpu.VMEM((NCHUNK, IDXW), jnp.int32),
        pltpu.VMEM((NCHUNK, IDXW), jnp.int32),
        pltpu.VMEM((BPW,), jnp.float32),
        pltpu.VMEM((BPW,), jnp.float32),
        pltpu.VMEM((BPW,), jnp.float32),
        pltpu.VMEM((BPW,), jnp.float32),
        pltpu.VMEM((BPW,), jnp.float32),
        pltpu.SemaphoreType.DMA,
    ],
)(_irt_body)


def kernel(student_id, exercise_id, theta_table, disc_table, diff_table, guess_table):
    sid = student_id.astype(jnp.int32).reshape(NW * NCHUNK, IDXW)
    eid = exercise_id.astype(jnp.int32).reshape(NW * NCHUNK, IDXW)
    return _irt(sid, eid,
                theta_table.reshape(-1), disc_table.reshape(-1),
                diff_table.reshape(-1), guess_table.reshape(-1))


# speedup vs baseline: 3.2515x; 2.7933x over previous
import functools

import jax
import jax.numpy as jnp
from jax import lax
from jax.experimental import pallas as pl
from jax.experimental.pallas import tpu as pltpu
from jax.experimental.pallas import tpu_sc as plsc

D = 1.702
NC = 2
NS = 16
NW = NC * NS
L = 16
B = 16384
BPW = B // NW
IDXW = 128
NCHUNK = BPW // IDXW


def _irt_body(sid_hbm, eid_hbm, theta_hbm, disc_hbm, diff_hbm, guess_hbm,
              out_hbm, sidx, eidx, th, di, df, gu, out_v, sem):
    wid = lax.axis_index("s") * NC + lax.axis_index("c")
    base = wid * BPW
    pltpu.sync_copy(sid_hbm.at[pl.ds(base, BPW)], sidx.at[0])
    pltpu.sync_copy(eid_hbm.at[pl.ds(base, BPW)], eidx.at[0])
    copies = []
    for j in range(NCHUNK):
        sl = pl.ds(j * IDXW, IDXW)
        copies.append(pltpu.async_copy(theta_hbm.at[sidx.at[:, sl]], th.at[:, sl], sem))
        copies.append(pltpu.async_copy(disc_hbm.at[eidx.at[:, sl]], di.at[:, sl], sem))
        copies.append(pltpu.async_copy(diff_hbm.at[eidx.at[:, sl]], df.at[:, sl], sem))
        copies.append(pltpu.async_copy(guess_hbm.at[eidx.at[:, sl]], gu.at[:, sl], sem))
    for c in copies:
        c.wait()
    for i in range(BPW // L):
        sl = pl.ds(i * L, L)
        x = di[0, sl]
        u = jnp.exp(-jnp.abs(x))
        s = u / (2.0 + u)
        s2 = s * s
        poly = 1.0 + s2 * (1.0 / 3.0 + s2 * (1.0 / 5.0 + s2 * (1.0 / 7.0 + s2 * (1.0 / 9.0))))
        sp = jnp.maximum(x, 0.0) + 2.0 * s * poly
        g = gu[0, sl]
        e = jnp.exp(-D * sp * (th[0, sl] - df[0, sl]))
        out_v[0, sl] = g + (1.0 - g) / (1.0 + e)
    pltpu.sync_copy(out_v.at[0], out_hbm.at[pl.ds(base, BPW)])


_irt = functools.partial(
    pl.kernel,
    out_type=jax.ShapeDtypeStruct((B,), jnp.float32),
    mesh=plsc.VectorSubcoreMesh(core_axis_name="c", subcore_axis_name="s"),
    scratch_types=[
        pltpu.VMEM((1, BPW), jnp.int32),
        pltpu.VMEM((1, BPW), jnp.int32),
        pltpu.VMEM((1, BPW), jnp.float32),
        pltpu.VMEM((1, BPW), jnp.float32),
        pltpu.VMEM((1, BPW), jnp.float32),
        pltpu.VMEM((1, BPW), jnp.float32),
        pltpu.VMEM((1, BPW), jnp.float32),
        pltpu.SemaphoreType.DMA,
    ],
)(_irt_body)


def kernel(student_id, exercise_id, theta_table, disc_table, diff_table, guess_table):
    return _irt(student_id.astype(jnp.int32), exercise_id.astype(jnp.int32),
                theta_table.reshape(1, -1), disc_table.reshape(1, -1),
                diff_table.reshape(1, -1), guess_table.reshape(1, -1))


# speedup vs baseline: 3.3182x; 1.0205x over previous
import functools

import jax
import jax.numpy as jnp
from jax import lax
from jax.experimental import pallas as pl
from jax.experimental.pallas import tpu as pltpu
from jax.experimental.pallas import tpu_sc as plsc

D = 1.702
NC = 2
NS = 16
NW = NC * NS
L = 16
B = 16384
BPW = B // NW
IDXW = 128
NCHUNK = BPW // IDXW


def _irt_body(sid_hbm, eid_hbm, theta_hbm, disc_hbm, diff_hbm, guess_hbm,
              out_hbm, sidx, eidx, th, di, df, gu, out_v, sem):
    wid = lax.axis_index("s") * NC + lax.axis_index("c")
    base = wid * BPW
    ic1 = pltpu.async_copy(sid_hbm.at[pl.ds(base, BPW)], sidx.at[0], sem)
    ic2 = pltpu.async_copy(eid_hbm.at[pl.ds(base, BPW)], eidx.at[0], sem)
    ic1.wait()
    ic2.wait()
    copies = [
        pltpu.async_copy(theta_hbm.at[sidx.at[:, :]], th.at[:, :], sem),
        pltpu.async_copy(disc_hbm.at[eidx.at[:, :]], di.at[:, :], sem),
        pltpu.async_copy(diff_hbm.at[eidx.at[:, :]], df.at[:, :], sem),
        pltpu.async_copy(guess_hbm.at[eidx.at[:, :]], gu.at[:, :], sem),
    ]
    for c in copies:
        c.wait()
    for i in range(BPW // L):
        sl = pl.ds(i * L, L)
        x = di[0, sl]
        u = jnp.exp(-jnp.abs(x))
        s = u / (2.0 + u)
        s2 = s * s
        poly = 1.0 + s2 * (1.0 / 3.0 + s2 * (1.0 / 5.0 + s2 * (1.0 / 7.0 + s2 * (1.0 / 9.0))))
        sp = jnp.maximum(x, 0.0) + 2.0 * s * poly
        g = gu[0, sl]
        e = jnp.exp(-D * sp * (th[0, sl] - df[0, sl]))
        out_v[0, sl] = g + (1.0 - g) / (1.0 + e)
    pltpu.sync_copy(out_v.at[0], out_hbm.at[pl.ds(base, BPW)])


_irt = functools.partial(
    pl.kernel,
    out_type=jax.ShapeDtypeStruct((B,), jnp.float32),
    mesh=plsc.VectorSubcoreMesh(core_axis_name="c", subcore_axis_name="s"),
    scratch_types=[
        pltpu.VMEM((1, BPW), jnp.int32),
        pltpu.VMEM((1, BPW), jnp.int32),
        pltpu.VMEM((1, BPW), jnp.float32),
        pltpu.VMEM((1, BPW), jnp.float32),
        pltpu.VMEM((1, BPW), jnp.float32),
        pltpu.VMEM((1, BPW), jnp.float32),
        pltpu.VMEM((1, BPW), jnp.float32),
        pltpu.SemaphoreType.DMA,
    ],
)(_irt_body)


def kernel(student_id, exercise_id, theta_table, disc_table, diff_table, guess_table):
    return _irt(student_id.astype(jnp.int32), exercise_id.astype(jnp.int32),
                theta_table.reshape(1, -1), disc_table.reshape(1, -1),
                diff_table.reshape(1, -1), guess_table.reshape(1, -1))


# speedup vs baseline: 3.4183x; 1.0301x over previous
import functools

import jax
import jax.numpy as jnp
from jax import lax
from jax.experimental import pallas as pl
from jax.experimental.pallas import tpu as pltpu
from jax.experimental.pallas import tpu_sc as plsc

D = 1.702
NC = 2
NS = 16
NW = NC * NS
L = 16
B = 16384
BPW = B // NW
IDXW = 128
NCHUNK = BPW // IDXW


def _irt_body(sid_hbm, eid_hbm, theta_hbm, disc_hbm, diff_hbm, guess_hbm,
              out_hbm, sidx, eidx, th, di, df, gu, out_v, sem, sem0, sem1):
    wid = lax.axis_index("s") * NC + lax.axis_index("c")
    base = wid * BPW
    ic1 = pltpu.async_copy(sid_hbm.at[pl.ds(base, BPW)], sidx.at[0], sem)
    ic2 = pltpu.async_copy(eid_hbm.at[pl.ds(base, BPW)], eidx.at[0], sem)
    H = BPW // 2
    h0 = pl.ds(0, H)
    h1 = pl.ds(H, H)
    ic1.wait()
    cth0 = pltpu.async_copy(theta_hbm.at[sidx.at[:, h0]], th.at[:, h0], sem0)
    cth1 = pltpu.async_copy(theta_hbm.at[sidx.at[:, h1]], th.at[:, h1], sem1)
    ic2.wait()
    cdi0 = pltpu.async_copy(disc_hbm.at[eidx.at[:, h0]], di.at[:, h0], sem0)
    cdf0 = pltpu.async_copy(diff_hbm.at[eidx.at[:, h0]], df.at[:, h0], sem0)
    cgu0 = pltpu.async_copy(guess_hbm.at[eidx.at[:, h0]], gu.at[:, h0], sem0)
    cdi1 = pltpu.async_copy(disc_hbm.at[eidx.at[:, h1]], di.at[:, h1], sem1)
    cdf1 = pltpu.async_copy(diff_hbm.at[eidx.at[:, h1]], df.at[:, h1], sem1)
    cgu1 = pltpu.async_copy(guess_hbm.at[eidx.at[:, h1]], gu.at[:, h1], sem1)

    def compute(i):
        sl = pl.ds(i * L, L)
        x = di[0, sl]
        u = jnp.exp(-jnp.abs(x))
        s = u / (2.0 + u)
        s2 = s * s
        poly = 1.0 + s2 * (1.0 / 3.0 + s2 * (1.0 / 5.0 + s2 * (1.0 / 7.0)))
        sp = jnp.maximum(x, 0.0) + 2.0 * s * poly
        g = gu[0, sl]
        e = jnp.exp(-D * sp * (th[0, sl] - df[0, sl]))
        out_v[0, sl] = g + (1.0 - g) / (1.0 + e)

    NG = BPW // L
    for c in (cth0, cdi0, cdf0, cgu0):
        c.wait()
    for i in range(NG // 2):
        compute(i)
    oc0 = pltpu.async_copy(out_v.at[0].at[h0], out_hbm.at[pl.ds(base, H)], sem)
    for c in (cth1, cdi1, cdf1, cgu1):
        c.wait()
    for i in range(NG // 2, NG):
        compute(i)
    oc1 = pltpu.async_copy(out_v.at[0].at[h1], out_hbm.at[pl.ds(base + H, H)], sem)
    oc0.wait()
    oc1.wait()


_irt = functools.partial(
    pl.kernel,
    out_type=jax.ShapeDtypeStruct((B,), jnp.float32),
    mesh=plsc.VectorSubcoreMesh(core_axis_name="c", subcore_axis_name="s"),
    scratch_types=[
        pltpu.VMEM((1, BPW), jnp.int32),
        pltpu.VMEM((1, BPW), jnp.int32),
        pltpu.VMEM((1, BPW), jnp.float32),
        pltpu.VMEM((1, BPW), jnp.float32),
        pltpu.VMEM((1, BPW), jnp.float32),
        pltpu.VMEM((1, BPW), jnp.float32),
        pltpu.VMEM((1, BPW), jnp.float32),
        pltpu.SemaphoreType.DMA,
        pltpu.SemaphoreType.DMA,
        pltpu.SemaphoreType.DMA,
    ],
)(_irt_body)


def kernel(student_id, exercise_id, theta_table, disc_table, diff_table, guess_table):
    return _irt(student_id.astype(jnp.int32), exercise_id.astype(jnp.int32),
                theta_table.reshape(1, -1), disc_table.reshape(1, -1),
                diff_table.reshape(1, -1), guess_table.reshape(1, -1))


# speedup vs baseline: 3.5506x; 1.0387x over previous
import functools

import jax
import jax.numpy as jnp
from jax import lax
from jax.experimental import pallas as pl
from jax.experimental.pallas import tpu as pltpu
from jax.experimental.pallas import tpu_sc as plsc

D = 1.702
NC = 2
NS = 16
NW = NC * NS
L = 16
B = 16384
BPW = B // NW
IDXW = 128
NCHUNK = BPW // IDXW


def _irt_body(sid_hbm, eid_hbm, theta_hbm, disc_hbm, diff_hbm, guess_hbm,
              out_hbm, sidx, eidx, th, di, df, gu, out_v, sem, sem0, sem1):
    wid = lax.axis_index("s") * NC + lax.axis_index("c")
    base = wid * BPW
    ic1 = pltpu.async_copy(sid_hbm.at[pl.ds(base, BPW)], sidx.at[0], sem)
    ic2 = pltpu.async_copy(eid_hbm.at[pl.ds(base, BPW)], eidx.at[0], sem)
    H = BPW // 2
    h0 = pl.ds(0, H)
    h1 = pl.ds(H, H)
    ic1.wait()
    cth0 = pltpu.async_copy(theta_hbm.at[sidx.at[:, h0]], th.at[:, h0], sem0)
    cth1 = pltpu.async_copy(theta_hbm.at[sidx.at[:, h1]], th.at[:, h1], sem1)
    ic2.wait()
    cdi0 = pltpu.async_copy(disc_hbm.at[eidx.at[:, h0]], di.at[:, h0], sem0)
    cdf0 = pltpu.async_copy(diff_hbm.at[eidx.at[:, h0]], df.at[:, h0], sem0)
    cgu0 = pltpu.async_copy(guess_hbm.at[eidx.at[:, h0]], gu.at[:, h0], sem0)
    cdi1 = pltpu.async_copy(disc_hbm.at[eidx.at[:, h1]], di.at[:, h1], sem1)
    cdf1 = pltpu.async_copy(diff_hbm.at[eidx.at[:, h1]], df.at[:, h1], sem1)
    cgu1 = pltpu.async_copy(guess_hbm.at[eidx.at[:, h1]], gu.at[:, h1], sem1)

    def compute_pair(i):
        sa = pl.ds((2 * i) * L, L)
        sb = pl.ds((2 * i + 1) * L, L)
        xa = di[0, sa]
        xb = di[0, sb]
        ua = jnp.exp(-jnp.abs(xa))
        ub = jnp.exp(-jnp.abs(xb))
        sa_ = ua / (2.0 + ua)
        sb_ = ub / (2.0 + ub)
        s2a = sa_ * sa_
        s2b = sb_ * sb_
        pa = 1.0 + s2a * (1.0 / 3.0 + s2a * (1.0 / 5.0 + s2a * (1.0 / 7.0)))
        pb = 1.0 + s2b * (1.0 / 3.0 + s2b * (1.0 / 5.0 + s2b * (1.0 / 7.0)))
        spa = jnp.maximum(xa, 0.0) + 2.0 * sa_ * pa
        spb = jnp.maximum(xb, 0.0) + 2.0 * sb_ * pb
        ta = th[0, sa] - df[0, sa]
        tb = th[0, sb] - df[0, sb]
        ea = jnp.exp(-D * spa * ta)
        eb = jnp.exp(-D * spb * tb)
        ga = gu[0, sa]
        gb = gu[0, sb]
        out_v[0, sa] = ga + (1.0 - ga) / (1.0 + ea)
        out_v[0, sb] = gb + (1.0 - gb) / (1.0 + eb)

    NP = BPW // (2 * L)
    for c in (cth0, cdi0, cdf0, cgu0):
        c.wait()
    for i in range(NP // 2):
        compute_pair(i)
    oc0 = pltpu.async_copy(out_v.at[0].at[h0], out_hbm.at[pl.ds(base, H)], sem)
    for c in (cth1, cdi1, cdf1, cgu1):
        c.wait()
    for i in range(NP // 2, NP):
        compute_pair(i)
    oc1 = pltpu.async_copy(out_v.at[0].at[h1], out_hbm.at[pl.ds(base + H, H)], sem)
    oc0.wait()
    oc1.wait()


_irt = functools.partial(
    pl.kernel,
    out_type=jax.ShapeDtypeStruct((B,), jnp.float32),
    mesh=plsc.VectorSubcoreMesh(core_axis_name="c", subcore_axis_name="s"),
    scratch_types=[
        pltpu.VMEM((1, BPW), jnp.int32),
        pltpu.VMEM((1, BPW), jnp.int32),
        pltpu.VMEM((1, BPW), jnp.float32),
        pltpu.VMEM((1, BPW), jnp.float32),
        pltpu.VMEM((1, BPW), jnp.float32),
        pltpu.VMEM((1, BPW), jnp.float32),
        pltpu.VMEM((1, BPW), jnp.float32),
        pltpu.SemaphoreType.DMA,
        pltpu.SemaphoreType.DMA,
        pltpu.SemaphoreType.DMA,
    ],
)(_irt_body)


def kernel(student_id, exercise_id, theta_table, disc_table, diff_table, guess_table):
    return _irt(student_id.astype(jnp.int32), exercise_id.astype(jnp.int32),
                theta_table.reshape(1, -1), disc_table.reshape(1, -1),
                diff_table.reshape(1, -1), guess_table.reshape(1, -1))


# speedup vs baseline: 3.7481x; 1.0556x over previous
import functools

import jax
import jax.numpy as jnp
from jax import lax
from jax.experimental import pallas as pl
from jax.experimental.pallas import tpu as pltpu
from jax.experimental.pallas import tpu_sc as plsc

D = 1.702
NC = 2
NS = 16
NW = NC * NS
L = 16
B = 16384
BPW = B // NW
IDXW = 128
NCHUNK = BPW // IDXW


def _irt_body(sid_hbm, eid_hbm, theta_hbm, disc_hbm, diff_hbm, guess_hbm,
              out_hbm, sidx, eidx, th, di, df, gu, out_v, sem, sem0, sem1):
    wid = lax.axis_index("s") * NC + lax.axis_index("c")
    base = wid * BPW
    ic1 = pltpu.async_copy(sid_hbm.at[pl.ds(base, BPW)], sidx.at[0], sem)
    ic2 = pltpu.async_copy(eid_hbm.at[pl.ds(base, BPW)], eidx.at[0], sem)
    H = BPW // 2
    h0 = pl.ds(0, H)
    h1 = pl.ds(H, H)
    ic1.wait()
    cth0 = pltpu.async_copy(theta_hbm.at[sidx.at[:, h0]], th.at[:, h0], sem0)
    cth1 = pltpu.async_copy(theta_hbm.at[sidx.at[:, h1]], th.at[:, h1], sem1)
    ic2.wait()
    cdi0 = pltpu.async_copy(disc_hbm.at[eidx.at[:, h0]], di.at[:, h0], sem0)
    cdf0 = pltpu.async_copy(diff_hbm.at[eidx.at[:, h0]], df.at[:, h0], sem0)
    cgu0 = pltpu.async_copy(guess_hbm.at[eidx.at[:, h0]], gu.at[:, h0], sem0)
    cdi1 = pltpu.async_copy(disc_hbm.at[eidx.at[:, h1]], di.at[:, h1], sem1)
    cdf1 = pltpu.async_copy(diff_hbm.at[eidx.at[:, h1]], df.at[:, h1], sem1)
    cgu1 = pltpu.async_copy(guess_hbm.at[eidx.at[:, h1]], gu.at[:, h1], sem1)

    def compute_pair(i):
        sa = pl.ds((2 * i) * L, L)
        sb = pl.ds((2 * i + 1) * L, L)
        xa = di[0, sa]
        xb = di[0, sb]
        ua = jnp.exp(-jnp.abs(xa))
        ub = jnp.exp(-jnp.abs(xb))
        sa_ = ua / (2.0 + ua)
        sb_ = ub / (2.0 + ub)
        s2a = sa_ * sa_
        s2b = sb_ * sb_
        pa = 1.0 + s2a * (1.0 / 3.0 + s2a * (1.0 / 5.0 + s2a * (1.0 / 7.0)))
        pb = 1.0 + s2b * (1.0 / 3.0 + s2b * (1.0 / 5.0 + s2b * (1.0 / 7.0)))
        spa = jnp.maximum(xa, 0.0) + 2.0 * sa_ * pa
        spb = jnp.maximum(xb, 0.0) + 2.0 * sb_ * pb
        ta = th[0, sa] - df[0, sa]
        tb = th[0, sb] - df[0, sb]
        ea = jnp.exp(-D * spa * ta)
        eb = jnp.exp(-D * spb * tb)
        ga = gu[0, sa]
        gb = gu[0, sb]
        out_v[0, sa] = ga + (1.0 - ga) / (1.0 + ea)
        out_v[0, sb] = gb + (1.0 - gb) / (1.0 + eb)

    NP = BPW // (2 * L)
    for c in (cth0, cdi0, cdf0, cgu0):
        c.wait()
    pl.loop(0, NP // 2)(compute_pair)
    oc0 = pltpu.async_copy(out_v.at[0].at[h0], out_hbm.at[pl.ds(base, H)], sem)
    for c in (cth1, cdi1, cdf1, cgu1):
        c.wait()
    pl.loop(NP // 2, NP)(compute_pair)
    oc1 = pltpu.async_copy(out_v.at[0].at[h1], out_hbm.at[pl.ds(base + H, H)], sem)
    oc0.wait()
    oc1.wait()


_irt = functools.partial(
    pl.kernel,
    out_type=jax.ShapeDtypeStruct((B,), jnp.float32),
    mesh=plsc.VectorSubcoreMesh(core_axis_name="c", subcore_axis_name="s"),
    scratch_types=[
        pltpu.VMEM((1, BPW), jnp.int32),
        pltpu.VMEM((1, BPW), jnp.int32),
        pltpu.VMEM((1, BPW), jnp.float32),
        pltpu.VMEM((1, BPW), jnp.float32),
        pltpu.VMEM((1, BPW), jnp.float32),
        pltpu.VMEM((1, BPW), jnp.float32),
        pltpu.VMEM((1, BPW), jnp.float32),
        pltpu.SemaphoreType.DMA,
        pltpu.SemaphoreType.DMA,
        pltpu.SemaphoreType.DMA,
    ],
)(_irt_body)


def kernel(student_id, exercise_id, theta_table, disc_table, diff_table, guess_table):
    return _irt(student_id.astype(jnp.int32), exercise_id.astype(jnp.int32),
                theta_table.reshape(1, -1), disc_table.reshape(1, -1),
                diff_table.reshape(1, -1), guess_table.reshape(1, -1))
